# baseline (speedup 1.0000x reference)
import functools

import jax
import jax.numpy as jnp
from jax import lax
from jax.experimental import pallas as pl
from jax.experimental.pallas import tpu as pltpu
from jax.experimental.pallas import tpu_sc as plsc

N = 10000
NP = 10240
E = 320000
B = 40
NBUF = 8
NS = 16
NC = 2
EP = 327680
ROWS_PER_SUB = NP // NS
NB_EDGE = EP // (NS * B)
NGRP = NB_EDGE // NBUF
EROWS = EP // B
R = 512
NBLK = NP // R


def _newton_rsqrt(v):
    i = lax.bitcast_convert_type(v, jnp.int32)
    y = lax.bitcast_convert_type(jnp.int32(0x5F3759DF) - (i >> 1), jnp.float32)
    for _ in range(3):
        y = y * (1.5 - 0.5 * v * y * y)
    return y


def _make_deg_kernel():
    mesh = plsc.VectorSubcoreMesh(core_axis_name="c", subcore_axis_name="s")

    @functools.partial(
        pl.kernel,
        out_type=jax.ShapeDtypeStruct((NP, 128), jnp.float32),
        mesh=mesh,
        scratch_types=[
            pltpu.VMEM((NBUF, B), jnp.int32),
            pltpu.VMEM((B,), jnp.float32),
            pltpu.VMEM((ROWS_PER_SUB,), jnp.float32),
            pltpu.VMEM((ROWS_PER_SUB, 128), jnp.float32),
            pltpu.VMEM_SHARED((NP,), jnp.float32),
        ] + [pltpu.SemaphoreType.DMA] * NBUF,
    )
    def deg_kernel(dst_hbm, dis_hbm, idx_v, ones_v, cnt_v, row_v, acc_sh, *ssem):
        cid = lax.axis_index("c")
        sid = lax.axis_index("s")

        @pl.when(cid == 0)
        def _():
            for k in range(B // 16):
                ones_v[pl.ds(k * 16, 16)] = jnp.ones((16,), jnp.float32)
            for k in range(ROWS_PER_SUB // 16):
                cnt_v[pl.ds(k * 16, 16)] = jnp.zeros((16,), jnp.float32)
            pltpu.sync_copy(cnt_v, acc_sh.at[pl.ds(sid * ROWS_PER_SUB, ROWS_PER_SUB)])
            plsc.subcore_barrier()

            eoff = sid * NB_EDGE

            def scat(b):
                return pltpu.make_async_copy(
                    ones_v, acc_sh.at[idx_v.at[b]], ssem[b])

            def body(t, carry):
                for b in range(NBUF):
                    @pl.when(t > 0)
                    def _w():
                        scat(b).wait()
                pltpu.sync_copy(dst_hbm.at[pl.ds(eoff + t * NBUF, NBUF)], idx_v)
                for b in range(NBUF):
                    pltpu.async_copy(ones_v, acc_sh.at[idx_v.at[b]], ssem[b],
                                     add=True)
                return carry

            lax.fori_loop(0, NGRP, body, 0)
            for b in range(NBUF):
                scat(b).wait()
            plsc.subcore_barrier()

            pltpu.sync_copy(acc_sh.at[pl.ds(sid * ROWS_PER_SUB, ROWS_PER_SUB)], cnt_v)
            for k in range(ROWS_PER_SUB // 16):
                v = cnt_v[pl.ds(k * 16, 16)] + 1.0
                cnt_v[pl.ds(k * 16, 16)] = _newton_rsqrt(v)

            def bcast(j, carry):
                v = cnt_v[pl.ds(j * 16, 16)]
                for lane in range(16):
                    row = jnp.full((16,), v[lane], jnp.float32)
                    for k in range(128 // 16):
                        row_v[j * 16 + lane, pl.ds(k * 16, 16)] = row
                return carry

            lax.fori_loop(0, ROWS_PER_SUB // 16, bcast, 0)
            pltpu.sync_copy(row_v, dis_hbm.at[pl.ds(sid * ROWS_PER_SUB, ROWS_PER_SUB)])

    return deg_kernel


_deg_kernel = _make_deg_kernel()


def _make_agg_kernel():
    mesh = plsc.VectorSubcoreMesh(core_axis_name="c", subcore_axis_name="s")

    H = NBUF // 2

    @functools.partial(
        pl.kernel,
        out_type=jax.ShapeDtypeStruct((2 * NP, 128), jnp.float32),
        mesh=mesh,
        scratch_types=[pltpu.VMEM((H, B), jnp.int32)] * 8
          + [pltpu.VMEM((B, 128), jnp.float32)] * NBUF
          + [pltpu.SemaphoreType.DMA] * (2 * NBUF + 2)
          + [pltpu.VMEM_SHARED((NP, 128), jnp.float32)],
    )
    def agg_kernel(src_hbm, dst_hbm, hp_hbm, out_hbm, *rest):
        idx = rest[:8]
        iset = (idx[0:4], idx[4:8])
        rows = rest[8:8 + NBUF]
        gsem = rest[8 + NBUF:8 + 2 * NBUF]
        ssem = rest[8 + 2 * NBUF:8 + 3 * NBUF]
        isem = rest[8 + 3 * NBUF:8 + 3 * NBUF + 2]
        acc_sh = rest[8 + 3 * NBUF + 2]
        cid = lax.axis_index("c")
        sid = lax.axis_index("s")
        rbase = sid * ROWS_PER_SUB

        pltpu.sync_copy(hp_hbm.at[pl.ds(cid * NP + rbase, ROWS_PER_SUB)],
                        acc_sh.at[pl.ds(rbase, ROWS_PER_SUB)])
        plsc.subcore_barrier()

        soff = cid * EROWS + sid * NB_EDGE
        doff = sid * NB_EDGE

        def idx_descs(p, g):
            sA, sB, dA, dB = iset[p]
            base = soff + g * NBUF
            dbase = doff + g * NBUF
            return [
                pltpu.make_async_copy(src_hbm.at[pl.ds(base, H)], sA, isem[p]),
                pltpu.make_async_copy(src_hbm.at[pl.ds(base + H, H)], sB, isem[p]),
                pltpu.make_async_copy(dst_hbm.at[pl.ds(dbase, H)], dA, isem[p]),
                pltpu.make_async_copy(dst_hbm.at[pl.ds(dbase + H, H)], dB, isem[p]),
            ]

        def scat(p, b):
            dref = iset[p][2] if b < H else iset[p][3]
            return pltpu.make_async_copy(
                rows[b], acc_sh.at[dref.at[b % H]], ssem[b])

        def process(g, p, first):
            sA, sB, dA, dB = iset[p]
            if not first:
                for b in range(H):
                    scat(1 - p, b).wait()
                for d in idx_descs(p, g):
                    d.wait()
            gdA = [pltpu.async_copy(hp_hbm.at[sA.at[b]], rows[b], gsem[b])
                   for b in range(H)]
            if not first:
                for b in range(H):
                    scat(1 - p, H + b).wait()
            gdB = [pltpu.async_copy(hp_hbm.at[sB.at[b]], rows[H + b],
                                    gsem[H + b]) for b in range(H)]
            gn = jnp.minimum(g + 1, NGRP - 1)
            for d in idx_descs(1 - p, gn):
                d.start()
            for b in range(H):
                gdA[b].wait()
                pltpu.async_copy(rows[b], acc_sh.at[dA.at[b]], ssem[b],
                                 add=True)
            for b in range(H):
                gdB[b].wait()
                pltpu.async_copy(rows[H + b], acc_sh.at[dB.at[b]],
                                 ssem[H + b], add=True)

        for d in idx_descs(0, 0):
            d.start()
        for d in idx_descs(0, 0):
            d.wait()
        process(0, 0, True)

        def body(tt, carry):
            g = 1 + tt * 2
            process(g, 1, False)
            process(g + 1, 0, False)
            return carry

        lax.fori_loop(0, (NGRP - 2) // 2, body, 0)
        process(NGRP - 1, 1, False)
        for b in range(NBUF):
            scat(1, b).wait()
        for d in idx_descs(0, NGRP - 1):
            d.wait()
        plsc.subcore_barrier()

        pltpu.sync_copy(acc_sh.at[pl.ds(rbase, ROWS_PER_SUB)],
                        out_hbm.at[pl.ds(cid * NP + rbase, ROWS_PER_SUB)])

    return agg_kernel


_agg_kernel = _make_agg_kernel()


def _tc1_body(x_ref, w_ref, dis_ref, o_ref):
    o_ref[...] = jnp.dot(x_ref[...], w_ref[...],
                         preferred_element_type=jnp.float32) * dis_ref[...]


def _tc1(xp, W1, dis):
    return pl.pallas_call(
        _tc1_body,
        out_shape=jax.ShapeDtypeStruct((2 * NP, 128), jnp.float32),
        grid=(NC, NBLK),
        in_specs=[
            pl.BlockSpec((R, 128), lambda c, i: (i, 0)),
            pl.BlockSpec((128, 128), lambda c, i: (0, c)),
            pl.BlockSpec((R, 128), lambda c, i: (i, 0)),
        ],
        out_specs=pl.BlockSpec((R, 128), lambda c, i: (c * NBLK + i, 0)),
    )(xp, W1, dis)


def _tcmid_body(s0_ref, s1_ref, dis_ref, b_ref, w0_ref, w1_ref, o_ref):
    dis = dis_ref[...]
    z0 = jnp.maximum(dis * s0_ref[...] + b_ref[0:1, :], 0.0)
    z1 = jnp.maximum(dis * s1_ref[...] + b_ref[1:2, :], 0.0)
    acc = jnp.dot(z0, w0_ref[...], preferred_element_type=jnp.float32)
    acc = acc + jnp.dot(z1, w1_ref[...], preferred_element_type=jnp.float32)
    o_ref[...] = acc * dis


def _tcmid(S, W, b, dis):
    b2d = b.reshape(2, 128)
    return pl.pallas_call(
        _tcmid_body,
        out_shape=jax.ShapeDtypeStruct((2 * NP, 128), jnp.float32),
        grid=(NC, NBLK),
        in_specs=[
            pl.BlockSpec((R, 128), lambda c, i: (i, 0)),
            pl.BlockSpec((R, 128), lambda c, i: (NBLK + i, 0)),
            pl.BlockSpec((R, 128), lambda c, i: (i, 0)),
            pl.BlockSpec((2, 128), lambda c, i: (0, 0)),
            pl.BlockSpec((128, 128), lambda c, i: (0, c)),
            pl.BlockSpec((128, 128), lambda c, i: (1, c)),
        ],
        out_specs=pl.BlockSpec((R, 128), lambda c, i: (c * NBLK + i, 0)),
    )(S, S, dis, b2d, W, W)


def _tcfc_body(s0_ref, s1_ref, dis_ref, b_ref, w0_ref, w1_ref,
               bfc_ref, o_ref):
    dis = dis_ref[...]
    z0 = jnp.maximum(dis * s0_ref[...] + b_ref[0:1, :], 0.0)
    z1 = jnp.maximum(dis * s1_ref[...] + b_ref[1:2, :], 0.0)
    acc = jnp.dot(z0, w0_ref[...], preferred_element_type=jnp.float32)
    acc = acc + jnp.dot(z1, w1_ref[...], preferred_element_type=jnp.float32)
    o_ref[...] = acc + bfc_ref[...]


def _tcfc(S, Wfc, b, bfc, dis):
    b2d = b.reshape(2, 128)
    return pl.pallas_call(
        _tcfc_body,
        out_shape=jax.ShapeDtypeStruct((NP, 128), jnp.float32),
        grid=(NBLK,),
        in_specs=[
            pl.BlockSpec((R, 128), lambda i: (i, 0)),
            pl.BlockSpec((R, 128), lambda i: (NBLK + i, 0)),
            pl.BlockSpec((R, 128), lambda i: (i, 0)),
            pl.BlockSpec((2, 128), lambda i: (0, 0)),
            pl.BlockSpec((128, 128), lambda i: (0, 0)),
            pl.BlockSpec((128, 128), lambda i: (1, 0)),
            pl.BlockSpec((1, 128), lambda i: (0, 0)),
        ],
        out_specs=pl.BlockSpec((R, 128), lambda i: (i, 0)),
    )(S, S, dis, b2d, Wfc, Wfc, bfc.reshape(1, 128))


def kernel(x, edge_index, W1, b1, W2, b2, W3, b3, Wfc, bfc):
    src = edge_index[0].astype(jnp.int32)
    dst = edge_index[1].astype(jnp.int32)
    pad = (jnp.arange(EP - E, dtype=jnp.int32) % 224) + 10016
    srcp = jnp.concatenate([src, pad])
    dstp = jnp.concatenate([dst, pad])
    src3 = jnp.concatenate([srcp, srcp + NP]).reshape(2 * EROWS, B)
    dst3 = dstp.reshape(EROWS, B)
    xp = jnp.zeros((NP, x.shape[1]), x.dtype).at[:N].set(x)

    dis = _deg_kernel(dst3)
    hp = _tc1(xp, W1, dis)
    S = _agg_kernel(src3, dst3, hp)
    hp = _tcmid(S, W2, b1, dis)
    S = _agg_kernel(src3, dst3, hp)
    hp = _tcmid(S, W3, b2, dis)
    S = _agg_kernel(src3, dst3, hp)
    out = _tcfc(S, Wfc, b3, bfc, dis)
    return out[:N]


# speedup vs baseline: 1.0251x; 1.0251x over previous
import functools

import jax
import jax.numpy as jnp
from jax import lax
from jax.experimental import pallas as pl
from jax.experimental.pallas import tpu as pltpu
from jax.experimental.pallas import tpu_sc as plsc

N = 10000
NP = 10240
E = 320000
B = 80
NBUF = 4
NS = 16
NC = 2
EP = 327680
ROWS_PER_SUB = NP // NS
NB_EDGE = EP // (NS * B)
NGRP = NB_EDGE // NBUF
EROWS = EP // B
R = 512
NBLK = NP // R


def _newton_rsqrt(v):
    i = lax.bitcast_convert_type(v, jnp.int32)
    y = lax.bitcast_convert_type(jnp.int32(0x5F3759DF) - (i >> 1), jnp.float32)
    for _ in range(3):
        y = y * (1.5 - 0.5 * v * y * y)
    return y


def _make_deg_kernel():
    mesh = plsc.VectorSubcoreMesh(core_axis_name="c", subcore_axis_name="s")

    @functools.partial(
        pl.kernel,
        out_type=jax.ShapeDtypeStruct((NP,), jnp.float32),
        mesh=mesh,
        scratch_types=[
            pltpu.VMEM((NBUF, B), jnp.int32),
            pltpu.VMEM((B,), jnp.float32),
            pltpu.VMEM((ROWS_PER_SUB,), jnp.float32),
            pltpu.VMEM_SHARED((NP,), jnp.float32),
        ] + [pltpu.SemaphoreType.DMA] * NBUF,
    )
    def deg_kernel(dst_hbm, dis_hbm, idx_v, ones_v, cnt_v, acc_sh, *ssem):
        cid = lax.axis_index("c")
        sid = lax.axis_index("s")

        @pl.when(cid == 0)
        def _():
            for k in range(B // 16):
                ones_v[pl.ds(k * 16, 16)] = jnp.ones((16,), jnp.float32)
            for k in range(ROWS_PER_SUB // 16):
                cnt_v[pl.ds(k * 16, 16)] = jnp.zeros((16,), jnp.float32)
            pltpu.sync_copy(cnt_v, acc_sh.at[pl.ds(sid * ROWS_PER_SUB, ROWS_PER_SUB)])
            plsc.subcore_barrier()

            eoff = sid * NB_EDGE

            def scat(b):
                return pltpu.make_async_copy(
                    ones_v, acc_sh.at[idx_v.at[b]], ssem[b])

            def body(t, carry):
                for b in range(NBUF):
                    @pl.when(t > 0)
                    def _w():
                        scat(b).wait()
                pltpu.sync_copy(dst_hbm.at[pl.ds(eoff + t * NBUF, NBUF)], idx_v)
                for b in range(NBUF):
                    pltpu.async_copy(ones_v, acc_sh.at[idx_v.at[b]], ssem[b],
                                     add=True)
                return carry

            lax.fori_loop(0, NGRP, body, 0)
            for b in range(NBUF):
                scat(b).wait()
            plsc.subcore_barrier()

            pltpu.sync_copy(acc_sh.at[pl.ds(sid * ROWS_PER_SUB, ROWS_PER_SUB)], cnt_v)
            for k in range(ROWS_PER_SUB // 16):
                v = cnt_v[pl.ds(k * 16, 16)] + 1.0
                cnt_v[pl.ds(k * 16, 16)] = _newton_rsqrt(v)

            pltpu.sync_copy(cnt_v, dis_hbm.at[pl.ds(sid * ROWS_PER_SUB, ROWS_PER_SUB)])

    return deg_kernel


_deg_kernel = _make_deg_kernel()


def _make_agg_kernel():
    mesh = plsc.VectorSubcoreMesh(core_axis_name="c", subcore_axis_name="s")

    H = NBUF // 2

    @functools.partial(
        pl.kernel,
        out_type=jax.ShapeDtypeStruct((2 * NP, 128), jnp.float32),
        mesh=mesh,
        scratch_types=[pltpu.VMEM((H, B), jnp.int32)] * 8
          + [pltpu.VMEM((B, 128), jnp.float32)] * NBUF
          + [pltpu.SemaphoreType.DMA] * (2 * NBUF + 2)
          + [pltpu.VMEM_SHARED((NP, 128), jnp.float32)],
    )
    def agg_kernel(src_hbm, dst_hbm, hp_hbm, out_hbm, *rest):
        idx = rest[:8]
        iset = (idx[0:4], idx[4:8])
        rows = rest[8:8 + NBUF]
        gsem = rest[8 + NBUF:8 + 2 * NBUF]
        ssem = rest[8 + 2 * NBUF:8 + 3 * NBUF]
        isem = rest[8 + 3 * NBUF:8 + 3 * NBUF + 2]
        acc_sh = rest[8 + 3 * NBUF + 2]
        cid = lax.axis_index("c")
        sid = lax.axis_index("s")
        rbase = sid * ROWS_PER_SUB

        pltpu.sync_copy(hp_hbm.at[pl.ds(cid * NP + rbase, ROWS_PER_SUB)],
                        acc_sh.at[pl.ds(rbase, ROWS_PER_SUB)])
        plsc.subcore_barrier()

        soff = cid * EROWS + sid * NB_EDGE
        doff = sid * NB_EDGE

        def idx_descs(p, g):
            sA, sB, dA, dB = iset[p]
            base = soff + g * NBUF
            dbase = doff + g * NBUF
            return [
                pltpu.make_async_copy(src_hbm.at[pl.ds(base, H)], sA, isem[p]),
                pltpu.make_async_copy(src_hbm.at[pl.ds(base + H, H)], sB, isem[p]),
                pltpu.make_async_copy(dst_hbm.at[pl.ds(dbase, H)], dA, isem[p]),
                pltpu.make_async_copy(dst_hbm.at[pl.ds(dbase + H, H)], dB, isem[p]),
            ]

        def scat(p, b):
            dref = iset[p][2] if b < H else iset[p][3]
            return pltpu.make_async_copy(
                rows[b], acc_sh.at[dref.at[b % H]], ssem[b])

        def process(g, p, first):
            sA, sB, dA, dB = iset[p]
            if not first:
                for b in range(H):
                    scat(1 - p, b).wait()
                for d in idx_descs(p, g):
                    d.wait()
            gdA = [pltpu.async_copy(hp_hbm.at[sA.at[b]], rows[b], gsem[b])
                   for b in range(H)]
            if not first:
                for b in range(H):
                    scat(1 - p, H + b).wait()
            gdB = [pltpu.async_copy(hp_hbm.at[sB.at[b]], rows[H + b],
                                    gsem[H + b]) for b in range(H)]
            gn = jnp.minimum(g + 1, NGRP - 1)
            for d in idx_descs(1 - p, gn):
                d.start()
            for b in range(H):
                gdA[b].wait()
                pltpu.async_copy(rows[b], acc_sh.at[dA.at[b]], ssem[b],
                                 add=True)
            for b in range(H):
                gdB[b].wait()
                pltpu.async_copy(rows[H + b], acc_sh.at[dB.at[b]],
                                 ssem[H + b], add=True)

        for d in idx_descs(0, 0):
            d.start()
        for d in idx_descs(0, 0):
            d.wait()
        process(0, 0, True)

        def body(tt, carry):
            g = 1 + tt * 2
            process(g, 1, False)
            process(g + 1, 0, False)
            return carry

        lax.fori_loop(0, (NGRP - 2) // 2, body, 0)
        process(NGRP - 1, 1, False)
        for b in range(NBUF):
            scat(1, b).wait()
        for d in idx_descs(0, NGRP - 1):
            d.wait()
        plsc.subcore_barrier()

        pltpu.sync_copy(acc_sh.at[pl.ds(rbase, ROWS_PER_SUB)],
                        out_hbm.at[pl.ds(cid * NP + rbase, ROWS_PER_SUB)])

    return agg_kernel


_agg_kernel = _make_agg_kernel()


def _tc1_body(x_ref, w_ref, dis_ref, o_ref):
    o_ref[...] = jnp.dot(x_ref[...], w_ref[...],
                         preferred_element_type=jnp.float32) * dis_ref[...]


def _tc1(xp, W1, dis):
    return pl.pallas_call(
        _tc1_body,
        out_shape=jax.ShapeDtypeStruct((2 * NP, 128), jnp.float32),
        grid=(NC, NBLK),
        in_specs=[
            pl.BlockSpec((R, 128), lambda c, i: (i, 0)),
            pl.BlockSpec((128, 128), lambda c, i: (0, c)),
            pl.BlockSpec((R, 1), lambda c, i: (i, 0)),
        ],
        out_specs=pl.BlockSpec((R, 128), lambda c, i: (c * NBLK + i, 0)),
    )(xp, W1, dis)


def _tcmid_body(s0_ref, s1_ref, dis_ref, b_ref, w0_ref, w1_ref, o_ref):
    dis = dis_ref[...]
    z0 = jnp.maximum(dis * s0_ref[...] + b_ref[0:1, :], 0.0)
    z1 = jnp.maximum(dis * s1_ref[...] + b_ref[1:2, :], 0.0)
    acc = jnp.dot(z0, w0_ref[...], preferred_element_type=jnp.float32)
    acc = acc + jnp.dot(z1, w1_ref[...], preferred_element_type=jnp.float32)
    o_ref[...] = acc * dis


def _tcmid(S, W, b, dis):
    b2d = b.reshape(2, 128)
    return pl.pallas_call(
        _tcmid_body,
        out_shape=jax.ShapeDtypeStruct((2 * NP, 128), jnp.float32),
        grid=(NC, NBLK),
        in_specs=[
            pl.BlockSpec((R, 128), lambda c, i: (i, 0)),
            pl.BlockSpec((R, 128), lambda c, i: (NBLK + i, 0)),
            pl.BlockSpec((R, 1), lambda c, i: (i, 0)),
            pl.BlockSpec((2, 128), lambda c, i: (0, 0)),
            pl.BlockSpec((128, 128), lambda c, i: (0, c)),
            pl.BlockSpec((128, 128), lambda c, i: (1, c)),
        ],
        out_specs=pl.BlockSpec((R, 128), lambda c, i: (c * NBLK + i, 0)),
    )(S, S, dis, b2d, W, W)


def _tcfc_body(s0_ref, s1_ref, dis_ref, b_ref, w0_ref, w1_ref,
               bfc_ref, o_ref):
    dis = dis_ref[...]
    z0 = jnp.maximum(dis * s0_ref[...] + b_ref[0:1, :], 0.0)
    z1 = jnp.maximum(dis * s1_ref[...] + b_ref[1:2, :], 0.0)
    acc = jnp.dot(z0, w0_ref[...], preferred_element_type=jnp.float32)
    acc = acc + jnp.dot(z1, w1_ref[...], preferred_element_type=jnp.float32)
    o_ref[...] = acc + bfc_ref[...]


def _tcfc(S, Wfc, b, bfc, dis):
    b2d = b.reshape(2, 128)
    return pl.pallas_call(
        _tcfc_body,
        out_shape=jax.ShapeDtypeStruct((NP, 128), jnp.float32),
        grid=(NBLK,),
        in_specs=[
            pl.BlockSpec((R, 128), lambda i: (i, 0)),
            pl.BlockSpec((R, 128), lambda i: (NBLK + i, 0)),
            pl.BlockSpec((R, 1), lambda i: (i, 0)),
            pl.BlockSpec((2, 128), lambda i: (0, 0)),
            pl.BlockSpec((128, 128), lambda i: (0, 0)),
            pl.BlockSpec((128, 128), lambda i: (1, 0)),
            pl.BlockSpec((1, 128), lambda i: (0, 0)),
        ],
        out_specs=pl.BlockSpec((R, 128), lambda i: (i, 0)),
    )(S, S, dis, b2d, Wfc, Wfc, bfc.reshape(1, 128))


def kernel(x, edge_index, W1, b1, W2, b2, W3, b3, Wfc, bfc):
    src = edge_index[0].astype(jnp.int32)
    dst = edge_index[1].astype(jnp.int32)
    pad = (jnp.arange(EP - E, dtype=jnp.int32) % 224) + 10016
    srcp = jnp.concatenate([src, pad])
    dstp = jnp.concatenate([dst, pad])
    src3 = jnp.concatenate([srcp, srcp + NP]).reshape(2 * EROWS, B)
    dst3 = dstp.reshape(EROWS, B)
    xp = jnp.zeros((NP, x.shape[1]), x.dtype).at[:N].set(x)

    dis = _deg_kernel(dst3).reshape(NP, 1)
    hp = _tc1(xp, W1, dis)
    S = _agg_kernel(src3, dst3, hp)
    hp = _tcmid(S, W2, b1, dis)
    S = _agg_kernel(src3, dst3, hp)
    hp = _tcmid(S, W3, b2, dis)
    S = _agg_kernel(src3, dst3, hp)
    out = _tcfc(S, Wfc, b3, bfc, dis)
    return out[:N]


# speedup vs baseline: 1.0486x; 1.0229x over previous
import functools

import jax
import jax.numpy as jnp
from jax import lax
from jax.experimental import pallas as pl
from jax.experimental.pallas import tpu as pltpu
from jax.experimental.pallas import tpu_sc as plsc

N = 10000
NP = 10240
E = 320000
B = 80
NBUF = 4
NS = 16
NC = 2
EP = 327680
ROWS_PER_SUB = NP // NS
NB_EDGE = EP // (NS * B)
NGRP = NB_EDGE // NBUF
EROWS = EP // B
R = 512
NBLK = NP // R


DNB = EROWS // (2 * NS)
DNG = DNB // NBUF


def _make_deg_kernel():
    mesh = plsc.VectorSubcoreMesh(core_axis_name="c", subcore_axis_name="s")

    @functools.partial(
        pl.kernel,
        out_type=jax.ShapeDtypeStruct((2 * NP,), jnp.float32),
        mesh=mesh,
        scratch_types=[
            pltpu.VMEM((NBUF, B), jnp.int32),
            pltpu.VMEM((NBUF, B), jnp.int32),
            pltpu.VMEM((B,), jnp.float32),
            pltpu.VMEM((ROWS_PER_SUB,), jnp.float32),
            pltpu.VMEM_SHARED((NP,), jnp.float32),
        ] + [pltpu.SemaphoreType.DMA] * (NBUF + 2),
    )
    def deg_kernel(dst_hbm, cnt_hbm, idx0, idx1, ones_v, z_v, acc_sh, *sems):
        ssem = sems[:NBUF]
        isem = sems[NBUF:NBUF + 2]
        iset = (idx0, idx1)
        cid = lax.axis_index("c")
        sid = lax.axis_index("s")

        for k in range(B // 16):
            ones_v[pl.ds(k * 16, 16)] = jnp.ones((16,), jnp.float32)
        for k in range(ROWS_PER_SUB // 16):
            z_v[pl.ds(k * 16, 16)] = jnp.zeros((16,), jnp.float32)
        pltpu.sync_copy(z_v, acc_sh.at[pl.ds(sid * ROWS_PER_SUB, ROWS_PER_SUB)])
        plsc.subcore_barrier()

        eoff = cid * (EROWS // 2) + sid * DNB

        def idx_desc(p, t):
            return pltpu.make_async_copy(
                dst_hbm.at[pl.ds(eoff + t * NBUF, NBUF)], iset[p], isem[p])

        def scat(p, b):
            return pltpu.make_async_copy(
                ones_v, acc_sh.at[iset[p].at[b]], ssem[b])

        def process(t, p, first):
            if not first:
                for b in range(NBUF):
                    scat(1 - p, b).wait()
                idx_desc(p, t).wait()
            tn = jnp.minimum(t + 1, DNG - 1)
            idx_desc(1 - p, tn).start()
            for b in range(NBUF):
                pltpu.async_copy(ones_v, acc_sh.at[iset[p].at[b]], ssem[b],
                                 add=True)

        idx_desc(0, 0).start()
        idx_desc(0, 0).wait()
        process(0, 0, True)

        def body(tt, carry):
            t = 1 + tt * 2
            process(t, 1, False)
            process(t + 1, 0, False)
            return carry

        lax.fori_loop(0, (DNG - 2) // 2, body, 0)
        process(DNG - 1, 1, False)
        for b in range(NBUF):
            scat(1, b).wait()
        idx_desc(0, DNG - 1).wait()
        plsc.subcore_barrier()

        pltpu.sync_copy(acc_sh.at[pl.ds(sid * ROWS_PER_SUB, ROWS_PER_SUB)],
                        cnt_hbm.at[pl.ds(cid * NP + sid * ROWS_PER_SUB,
                                         ROWS_PER_SUB)])

    return deg_kernel


_deg_kernel = _make_deg_kernel()


def _make_agg_kernel():
    mesh = plsc.VectorSubcoreMesh(core_axis_name="c", subcore_axis_name="s")

    H = NBUF // 2

    @functools.partial(
        pl.kernel,
        out_type=jax.ShapeDtypeStruct((2 * NP, 128), jnp.float32),
        mesh=mesh,
        scratch_types=[pltpu.VMEM((H, B), jnp.int32)] * 8
          + [pltpu.VMEM((B, 128), jnp.float32)] * NBUF
          + [pltpu.SemaphoreType.DMA] * (2 * NBUF + 2)
          + [pltpu.VMEM_SHARED((NP, 128), jnp.float32)],
    )
    def agg_kernel(src_hbm, dst_hbm, hp_hbm, out_hbm, *rest):
        idx = rest[:8]
        iset = (idx[0:4], idx[4:8])
        rows = rest[8:8 + NBUF]
        gsem = rest[8 + NBUF:8 + 2 * NBUF]
        ssem = rest[8 + 2 * NBUF:8 + 3 * NBUF]
        isem = rest[8 + 3 * NBUF:8 + 3 * NBUF + 2]
        acc_sh = rest[8 + 3 * NBUF + 2]
        cid = lax.axis_index("c")
        sid = lax.axis_index("s")
        rbase = sid * ROWS_PER_SUB

        pltpu.sync_copy(hp_hbm.at[pl.ds(cid * NP + rbase, ROWS_PER_SUB)],
                        acc_sh.at[pl.ds(rbase, ROWS_PER_SUB)])
        plsc.subcore_barrier()

        soff = cid * EROWS + sid * NB_EDGE
        doff = sid * NB_EDGE

        def idx_descs(p, g):
            sA, sB, dA, dB = iset[p]
            base = soff + g * NBUF
            dbase = doff + g * NBUF
            return [
                pltpu.make_async_copy(src_hbm.at[pl.ds(base, H)], sA, isem[p]),
                pltpu.make_async_copy(src_hbm.at[pl.ds(base + H, H)], sB, isem[p]),
                pltpu.make_async_copy(dst_hbm.at[pl.ds(dbase, H)], dA, isem[p]),
                pltpu.make_async_copy(dst_hbm.at[pl.ds(dbase + H, H)], dB, isem[p]),
            ]

        def scat(p, b):
            dref = iset[p][2] if b < H else iset[p][3]
            return pltpu.make_async_copy(
                rows[b], acc_sh.at[dref.at[b % H]], ssem[b])

        def process(g, p, first):
            sA, sB, dA, dB = iset[p]
            if not first:
                for b in range(H):
                    scat(1 - p, b).wait()
                for d in idx_descs(p, g):
                    d.wait()
            gdA = [pltpu.async_copy(hp_hbm.at[sA.at[b]], rows[b], gsem[b])
                   for b in range(H)]
            if not first:
                for b in range(H):
                    scat(1 - p, H + b).wait()
            gdB = [pltpu.async_copy(hp_hbm.at[sB.at[b]], rows[H + b],
                                    gsem[H + b]) for b in range(H)]
            gn = jnp.minimum(g + 1, NGRP - 1)
            for d in idx_descs(1 - p, gn):
                d.start()
            for b in range(H):
                gdA[b].wait()
                pltpu.async_copy(rows[b], acc_sh.at[dA.at[b]], ssem[b],
                                 add=True)
            for b in range(H):
                gdB[b].wait()
                pltpu.async_copy(rows[H + b], acc_sh.at[dB.at[b]],
                                 ssem[H + b], add=True)

        for d in idx_descs(0, 0):
            d.start()
        for d in idx_descs(0, 0):
            d.wait()
        process(0, 0, True)

        def body(tt, carry):
            g = 1 + tt * 2
            process(g, 1, False)
            process(g + 1, 0, False)
            return carry

        lax.fori_loop(0, (NGRP - 2) // 2, body, 0)
        process(NGRP - 1, 1, False)
        for b in range(NBUF):
            scat(1, b).wait()
        for d in idx_descs(0, NGRP - 1):
            d.wait()
        plsc.subcore_barrier()

        pltpu.sync_copy(acc_sh.at[pl.ds(rbase, ROWS_PER_SUB)],
                        out_hbm.at[pl.ds(cid * NP + rbase, ROWS_PER_SUB)])

    return agg_kernel


_agg_kernel = _make_agg_kernel()


def _dis(d0_ref, d1_ref):
    return lax.rsqrt(d0_ref[...] + d1_ref[...] + 1.0)


def _tc1_body(x_ref, w_ref, d0_ref, d1_ref, o_ref):
    o_ref[...] = jnp.dot(x_ref[...], w_ref[...],
                         preferred_element_type=jnp.float32) * _dis(d0_ref,
                                                                    d1_ref)


def _tc1(xp, W1, dis):
    return pl.pallas_call(
        _tc1_body,
        out_shape=jax.ShapeDtypeStruct((2 * NP, 128), jnp.float32),
        grid=(NC, NBLK),
        in_specs=[
            pl.BlockSpec((R, 128), lambda c, i: (i, 0)),
            pl.BlockSpec((128, 128), lambda c, i: (0, c)),
            pl.BlockSpec((R, 1), lambda c, i: (i, 0)),
            pl.BlockSpec((R, 1), lambda c, i: (NBLK + i, 0)),
        ],
        out_specs=pl.BlockSpec((R, 128), lambda c, i: (c * NBLK + i, 0)),
    )(xp, W1, dis, dis)


def _tcmid_body(s0_ref, s1_ref, d0_ref, d1_ref, b_ref, w0_ref, w1_ref, o_ref):
    dis = _dis(d0_ref, d1_ref)
    z0 = jnp.maximum(dis * s0_ref[...] + b_ref[0:1, :], 0.0)
    z1 = jnp.maximum(dis * s1_ref[...] + b_ref[1:2, :], 0.0)
    acc = jnp.dot(z0, w0_ref[...], preferred_element_type=jnp.float32)
    acc = acc + jnp.dot(z1, w1_ref[...], preferred_element_type=jnp.float32)
    o_ref[...] = acc * dis


def _tcmid(S, W, b, dis):
    b2d = b.reshape(2, 128)
    return pl.pallas_call(
        _tcmid_body,
        out_shape=jax.ShapeDtypeStruct((2 * NP, 128), jnp.float32),
        grid=(NC, NBLK),
        in_specs=[
            pl.BlockSpec((R, 128), lambda c, i: (i, 0)),
            pl.BlockSpec((R, 128), lambda c, i: (NBLK + i, 0)),
            pl.BlockSpec((R, 1), lambda c, i: (i, 0)),
            pl.BlockSpec((R, 1), lambda c, i: (NBLK + i, 0)),
            pl.BlockSpec((2, 128), lambda c, i: (0, 0)),
            pl.BlockSpec((128, 128), lambda c, i: (0, c)),
            pl.BlockSpec((128, 128), lambda c, i: (1, c)),
        ],
        out_specs=pl.BlockSpec((R, 128), lambda c, i: (c * NBLK + i, 0)),
    )(S, S, dis, dis, b2d, W, W)


def _tcfc_body(s0_ref, s1_ref, d0_ref, d1_ref, b_ref, w0_ref, w1_ref,
               bfc_ref, o_ref):
    dis = _dis(d0_ref, d1_ref)
    z0 = jnp.maximum(dis * s0_ref[...] + b_ref[0:1, :], 0.0)
    z1 = jnp.maximum(dis * s1_ref[...] + b_ref[1:2, :], 0.0)
    acc = jnp.dot(z0, w0_ref[...], preferred_element_type=jnp.float32)
    acc = acc + jnp.dot(z1, w1_ref[...], preferred_element_type=jnp.float32)
    o_ref[...] = acc + bfc_ref[...]


def _tcfc(S, Wfc, b, bfc, dis):
    b2d = b.reshape(2, 128)
    return pl.pallas_call(
        _tcfc_body,
        out_shape=jax.ShapeDtypeStruct((NP, 128), jnp.float32),
        grid=(NBLK,),
        in_specs=[
            pl.BlockSpec((R, 128), lambda i: (i, 0)),
            pl.BlockSpec((R, 128), lambda i: (NBLK + i, 0)),
            pl.BlockSpec((R, 1), lambda i: (i, 0)),
            pl.BlockSpec((R, 1), lambda i: (NBLK + i, 0)),
            pl.BlockSpec((2, 128), lambda i: (0, 0)),
            pl.BlockSpec((128, 128), lambda i: (0, 0)),
            pl.BlockSpec((128, 128), lambda i: (1, 0)),
            pl.BlockSpec((1, 128), lambda i: (0, 0)),
        ],
        out_specs=pl.BlockSpec((R, 128), lambda i: (i, 0)),
    )(S, S, dis, dis, b2d, Wfc, Wfc, bfc.reshape(1, 128))


def kernel(x, edge_index, W1, b1, W2, b2, W3, b3, Wfc, bfc):
    src = edge_index[0].astype(jnp.int32)
    dst = edge_index[1].astype(jnp.int32)
    pad = (jnp.arange(EP - E, dtype=jnp.int32) % 224) + 10016
    srcp = jnp.concatenate([src, pad])
    dstp = jnp.concatenate([dst, pad])
    src3 = jnp.concatenate([srcp, srcp + NP]).reshape(2 * EROWS, B)
    dst3 = dstp.reshape(EROWS, B)
    xp = jnp.zeros((NP, x.shape[1]), x.dtype).at[:N].set(x)

    dis = _deg_kernel(dst3).reshape(2 * NP, 1)
    hp = _tc1(xp, W1, dis)
    S = _agg_kernel(src3, dst3, hp)
    hp = _tcmid(S, W2, b1, dis)
    S = _agg_kernel(src3, dst3, hp)
    hp = _tcmid(S, W3, b2, dis)
    S = _agg_kernel(src3, dst3, hp)
    out = _tcfc(S, Wfc, b3, bfc, dis)
    return out[:N]


# speedup vs baseline: 1.0979x; 1.0470x over previous
import functools

import jax
import jax.numpy as jnp
from jax import lax
from jax.experimental import pallas as pl
from jax.experimental.pallas import tpu as pltpu
from jax.experimental.pallas import tpu_sc as plsc

N = 10000
NP = 10240
E = 320000
B = 80
NBUF = 4
NS = 16
NC = 2
EP = 327680
ROWS_PER_SUB = NP // NS
NB_EDGE = EP // (NS * B)
NGRP = NB_EDGE // NBUF
EROWS = EP // B
R = 1024
NBLK = NP // R


DNB = EROWS // (2 * NS)
DNG = DNB // NBUF


def _make_deg_kernel():
    mesh = plsc.VectorSubcoreMesh(core_axis_name="c", subcore_axis_name="s")

    @functools.partial(
        pl.kernel,
        out_type=jax.ShapeDtypeStruct((2 * NP,), jnp.float32),
        mesh=mesh,
        scratch_types=[
            pltpu.VMEM((NBUF, B), jnp.int32),
            pltpu.VMEM((NBUF, B), jnp.int32),
            pltpu.VMEM((B,), jnp.float32),
            pltpu.VMEM((ROWS_PER_SUB,), jnp.float32),
            pltpu.VMEM_SHARED((NP,), jnp.float32),
        ] + [pltpu.SemaphoreType.DMA] * (NBUF + 2),
    )
    def deg_kernel(dst_hbm, cnt_hbm, idx0, idx1, ones_v, z_v, acc_sh, *sems):
        ssem = sems[:NBUF]
        isem = sems[NBUF:NBUF + 2]
        iset = (idx0, idx1)
        cid = lax.axis_index("c")
        sid = lax.axis_index("s")

        for k in range(B // 16):
            ones_v[pl.ds(k * 16, 16)] = jnp.ones((16,), jnp.float32)
        for k in range(ROWS_PER_SUB // 16):
            z_v[pl.ds(k * 16, 16)] = jnp.zeros((16,), jnp.float32)
        pltpu.sync_copy(z_v, acc_sh.at[pl.ds(sid * ROWS_PER_SUB, ROWS_PER_SUB)])
        plsc.subcore_barrier()

        eoff = cid * (EROWS // 2) + sid * DNB

        def idx_desc(p, t):
            return pltpu.make_async_copy(
                dst_hbm.at[pl.ds(eoff + t * NBUF, NBUF)], iset[p], isem[p])

        def scat(p, b):
            return pltpu.make_async_copy(
                ones_v, acc_sh.at[iset[p].at[b]], ssem[b])

        def process(t, p, first):
            if not first:
                for b in range(NBUF):
                    scat(1 - p, b).wait()
                idx_desc(p, t).wait()
            tn = jnp.minimum(t + 1, DNG - 1)
            idx_desc(1 - p, tn).start()
            for b in range(NBUF):
                pltpu.async_copy(ones_v, acc_sh.at[iset[p].at[b]], ssem[b],
                                 add=True)

        idx_desc(0, 0).start()
        idx_desc(0, 0).wait()
        process(0, 0, True)

        def body(tt, carry):
            t = 1 + tt * 2
            process(t, 1, False)
            process(t + 1, 0, False)
            return carry

        lax.fori_loop(0, (DNG - 2) // 2, body, 0)
        process(DNG - 1, 1, False)
        for b in range(NBUF):
            scat(1, b).wait()
        idx_desc(0, DNG - 1).wait()
        plsc.subcore_barrier()

        pltpu.sync_copy(acc_sh.at[pl.ds(sid * ROWS_PER_SUB, ROWS_PER_SUB)],
                        cnt_hbm.at[pl.ds(cid * NP + sid * ROWS_PER_SUB,
                                         ROWS_PER_SUB)])

    return deg_kernel


_deg_kernel = _make_deg_kernel()


def _make_agg_kernel():
    mesh = plsc.VectorSubcoreMesh(core_axis_name="c", subcore_axis_name="s")

    H = NBUF // 2

    @functools.partial(
        pl.kernel,
        out_type=jax.ShapeDtypeStruct((2 * NP, 128), jnp.float32),
        mesh=mesh,
        scratch_types=[pltpu.VMEM((H, B), jnp.int32)] * 8
          + [pltpu.VMEM((B, 128), jnp.float32)] * NBUF
          + [pltpu.SemaphoreType.DMA] * (2 * NBUF + 2)
          + [pltpu.VMEM_SHARED((NP, 128), jnp.float32)],
    )
    def agg_kernel(src_hbm, dst_hbm, hp_hbm, out_hbm, *rest):
        idx = rest[:8]
        iset = (idx[0:4], idx[4:8])
        rows = rest[8:8 + NBUF]
        gsem = rest[8 + NBUF:8 + 2 * NBUF]
        ssem = rest[8 + 2 * NBUF:8 + 3 * NBUF]
        isem = rest[8 + 3 * NBUF:8 + 3 * NBUF + 2]
        acc_sh = rest[8 + 3 * NBUF + 2]
        cid = lax.axis_index("c")
        sid = lax.axis_index("s")
        rbase = sid * ROWS_PER_SUB

        pltpu.sync_copy(hp_hbm.at[pl.ds(cid * NP + rbase, ROWS_PER_SUB)],
                        acc_sh.at[pl.ds(rbase, ROWS_PER_SUB)])
        plsc.subcore_barrier()

        soff = cid * EROWS + sid * NB_EDGE
        doff = sid * NB_EDGE

        def idx_descs(p, g):
            sA, sB, dA, dB = iset[p]
            base = soff + g * NBUF
            dbase = doff + g * NBUF
            return [
                pltpu.make_async_copy(src_hbm.at[pl.ds(base, H)], sA, isem[p]),
                pltpu.make_async_copy(src_hbm.at[pl.ds(base + H, H)], sB, isem[p]),
                pltpu.make_async_copy(dst_hbm.at[pl.ds(dbase, H)], dA, isem[p]),
                pltpu.make_async_copy(dst_hbm.at[pl.ds(dbase + H, H)], dB, isem[p]),
            ]

        def scat(p, b):
            dref = iset[p][2] if b < H else iset[p][3]
            return pltpu.make_async_copy(
                rows[b], acc_sh.at[dref.at[b % H]], ssem[b])

        def process(g, p, first):
            sA, sB, dA, dB = iset[p]
            if not first:
                for b in range(H):
                    scat(1 - p, b).wait()
                for d in idx_descs(p, g):
                    d.wait()
            gdA = [pltpu.async_copy(hp_hbm.at[sA.at[b]], rows[b], gsem[b])
                   for b in range(H)]
            if not first:
                for b in range(H):
                    scat(1 - p, H + b).wait()
            gdB = [pltpu.async_copy(hp_hbm.at[sB.at[b]], rows[H + b],
                                    gsem[H + b]) for b in range(H)]
            gn = jnp.minimum(g + 1, NGRP - 1)
            for d in idx_descs(1 - p, gn):
                d.start()
            for b in range(H):
                gdA[b].wait()
                pltpu.async_copy(rows[b], acc_sh.at[dA.at[b]], ssem[b],
                                 add=True)
            for b in range(H):
                gdB[b].wait()
                pltpu.async_copy(rows[H + b], acc_sh.at[dB.at[b]],
                                 ssem[H + b], add=True)

        for d in idx_descs(0, 0):
            d.start()
        for d in idx_descs(0, 0):
            d.wait()
        process(0, 0, True)

        def body(tt, carry):
            g = 1 + tt * 2
            process(g, 1, False)
            process(g + 1, 0, False)
            return carry

        lax.fori_loop(0, (NGRP - 2) // 2, body, 0)
        process(NGRP - 1, 1, False)
        for b in range(NBUF):
            scat(1, b).wait()
        for d in idx_descs(0, NGRP - 1):
            d.wait()
        plsc.subcore_barrier()

        pltpu.sync_copy(acc_sh.at[pl.ds(rbase, ROWS_PER_SUB)],
                        out_hbm.at[pl.ds(cid * NP + rbase, ROWS_PER_SUB)])

    return agg_kernel


_agg_kernel = _make_agg_kernel()


def _dis(d0_ref, d1_ref):
    return lax.rsqrt(d0_ref[...] + d1_ref[...] + 1.0)


def _tc1_body(x_ref, w_ref, d0_ref, d1_ref, o_ref):
    o_ref[...] = jnp.dot(x_ref[...], w_ref[...],
                         preferred_element_type=jnp.float32) * _dis(d0_ref,
                                                                    d1_ref)


def _tc1(xp, W1, dis):
    return pl.pallas_call(
        _tc1_body,
        out_shape=jax.ShapeDtypeStruct((2 * NP, 128), jnp.float32),
        grid=(NC, NBLK),
        in_specs=[
            pl.BlockSpec((R, 128), lambda c, i: (i, 0)),
            pl.BlockSpec((128, 128), lambda c, i: (0, c)),
            pl.BlockSpec((R, 1), lambda c, i: (i, 0)),
            pl.BlockSpec((R, 1), lambda c, i: (NBLK + i, 0)),
        ],
        out_specs=pl.BlockSpec((R, 128), lambda c, i: (c * NBLK + i, 0)),
    )(xp, W1, dis, dis)


def _tcmid_body(s0_ref, s1_ref, d0_ref, d1_ref, b_ref, w0_ref, w1_ref, o_ref):
    dis = _dis(d0_ref, d1_ref)
    z0 = jnp.maximum(dis * s0_ref[...] + b_ref[0:1, :], 0.0)
    z1 = jnp.maximum(dis * s1_ref[...] + b_ref[1:2, :], 0.0)
    acc = jnp.dot(z0, w0_ref[...], preferred_element_type=jnp.float32)
    acc = acc + jnp.dot(z1, w1_ref[...], preferred_element_type=jnp.float32)
    o_ref[...] = acc * dis


def _tcmid(S, W, b, dis):
    b2d = b.reshape(2, 128)
    return pl.pallas_call(
        _tcmid_body,
        out_shape=jax.ShapeDtypeStruct((2 * NP, 128), jnp.float32),
        grid=(NC, NBLK),
        in_specs=[
            pl.BlockSpec((R, 128), lambda c, i: (i, 0)),
            pl.BlockSpec((R, 128), lambda c, i: (NBLK + i, 0)),
            pl.BlockSpec((R, 1), lambda c, i: (i, 0)),
            pl.BlockSpec((R, 1), lambda c, i: (NBLK + i, 0)),
            pl.BlockSpec((2, 128), lambda c, i: (0, 0)),
            pl.BlockSpec((128, 128), lambda c, i: (0, c)),
            pl.BlockSpec((128, 128), lambda c, i: (1, c)),
        ],
        out_specs=pl.BlockSpec((R, 128), lambda c, i: (c * NBLK + i, 0)),
    )(S, S, dis, dis, b2d, W, W)


def _tcfc_body(s0_ref, s1_ref, d0_ref, d1_ref, b_ref, w0_ref, w1_ref,
               bfc_ref, o_ref):
    dis = _dis(d0_ref, d1_ref)
    z0 = jnp.maximum(dis * s0_ref[...] + b_ref[0:1, :], 0.0)
    z1 = jnp.maximum(dis * s1_ref[...] + b_ref[1:2, :], 0.0)
    acc = jnp.dot(z0, w0_ref[...], preferred_element_type=jnp.float32)
    acc = acc + jnp.dot(z1, w1_ref[...], preferred_element_type=jnp.float32)
    o_ref[...] = acc + bfc_ref[...]


def _tcfc(S, Wfc, b, bfc, dis):
    b2d = b.reshape(2, 128)
    return pl.pallas_call(
        _tcfc_body,
        out_shape=jax.ShapeDtypeStruct((NP, 128), jnp.float32),
        grid=(NBLK,),
        in_specs=[
            pl.BlockSpec((R, 128), lambda i: (i, 0)),
            pl.BlockSpec((R, 128), lambda i: (NBLK + i, 0)),
            pl.BlockSpec((R, 1), lambda i: (i, 0)),
            pl.BlockSpec((R, 1), lambda i: (NBLK + i, 0)),
            pl.BlockSpec((2, 128), lambda i: (0, 0)),
            pl.BlockSpec((128, 128), lambda i: (0, 0)),
            pl.BlockSpec((128, 128), lambda i: (1, 0)),
            pl.BlockSpec((1, 128), lambda i: (0, 0)),
        ],
        out_specs=pl.BlockSpec((R, 128), lambda i: (i, 0)),
    )(S, S, dis, dis, b2d, Wfc, Wfc, bfc.reshape(1, 128))


def kernel(x, edge_index, W1, b1, W2, b2, W3, b3, Wfc, bfc):
    src = edge_index[0].astype(jnp.int32)
    dst = edge_index[1].astype(jnp.int32)
    pad = (jnp.arange(EP - E, dtype=jnp.int32) % 224) + 10016
    srcp = jnp.concatenate([src, pad])
    dstp = jnp.concatenate([dst, pad])
    src3 = jnp.concatenate([srcp, srcp + NP]).reshape(2 * EROWS, B)
    dst3 = dstp.reshape(EROWS, B)
    xp = jnp.zeros((NP, x.shape[1]), x.dtype).at[:N].set(x)

    dis = _deg_kernel(dst3).reshape(2 * NP, 1)
    hp = _tc1(xp, W1, dis)
    S = _agg_kernel(src3, dst3, hp)
    hp = _tcmid(S, W2, b1, dis)
    S = _agg_kernel(src3, dst3, hp)
    hp = _tcmid(S, W3, b2, dis)
    S = _agg_kernel(src3, dst3, hp)
    out = _tcfc(S, Wfc, b3, bfc, dis)
    return out[:N]


# speedup vs baseline: 1.1242x; 1.0240x over previous
import functools

import jax
import jax.numpy as jnp
from jax import lax
from jax.experimental import pallas as pl
from jax.experimental.pallas import tpu as pltpu
from jax.experimental.pallas import tpu_sc as plsc

N = 10000
NP = 10240
E = 320000
B = 80
NBUF = 4
NS = 16
NC = 2
EP = 327680
ROWS_PER_SUB = NP // NS
NB_EDGE = EP // (NS * B)
NGRP = NB_EDGE // NBUF
EROWS = EP // B
R = 2048
NBLK = NP // R


DNB = EROWS // (2 * NS)
DNG = DNB // NBUF


def _make_deg_kernel():
    mesh = plsc.VectorSubcoreMesh(core_axis_name="c", subcore_axis_name="s")

    @functools.partial(
        pl.kernel,
        out_type=jax.ShapeDtypeStruct((2 * NP,), jnp.float32),
        mesh=mesh,
        scratch_types=[
            pltpu.VMEM((NBUF, B), jnp.int32),
            pltpu.VMEM((NBUF, B), jnp.int32),
            pltpu.VMEM((B,), jnp.float32),
            pltpu.VMEM((ROWS_PER_SUB,), jnp.float32),
            pltpu.VMEM_SHARED((NP,), jnp.float32),
        ] + [pltpu.SemaphoreType.DMA] * (NBUF + 2),
    )
    def deg_kernel(dst_hbm, cnt_hbm, idx0, idx1, ones_v, z_v, acc_sh, *sems):
        ssem = sems[:NBUF]
        isem = sems[NBUF:NBUF + 2]
        iset = (idx0, idx1)
        cid = lax.axis_index("c")
        sid = lax.axis_index("s")

        for k in range(B // 16):
            ones_v[pl.ds(k * 16, 16)] = jnp.ones((16,), jnp.float32)
        for k in range(ROWS_PER_SUB // 16):
            z_v[pl.ds(k * 16, 16)] = jnp.zeros((16,), jnp.float32)
        pltpu.sync_copy(z_v, acc_sh.at[pl.ds(sid * ROWS_PER_SUB, ROWS_PER_SUB)])
        plsc.subcore_barrier()

        eoff = cid * (EROWS // 2) + sid * DNB

        def idx_desc(p, t):
            return pltpu.make_async_copy(
                dst_hbm.at[pl.ds(eoff + t * NBUF, NBUF)], iset[p], isem[p])

        def scat(p, b):
            return pltpu.make_async_copy(
                ones_v, acc_sh.at[iset[p].at[b]], ssem[b])

        def process(t, p, first):
            if not first:
                for b in range(NBUF):
                    scat(1 - p, b).wait()
                idx_desc(p, t).wait()
            tn = jnp.minimum(t + 1, DNG - 1)
            idx_desc(1 - p, tn).start()
            for b in range(NBUF):
                pltpu.async_copy(ones_v, acc_sh.at[iset[p].at[b]], ssem[b],
                                 add=True)

        idx_desc(0, 0).start()
        idx_desc(0, 0).wait()
        process(0, 0, True)

        def body(tt, carry):
            t = 1 + tt * 2
            process(t, 1, False)
            process(t + 1, 0, False)
            return carry

        lax.fori_loop(0, (DNG - 2) // 2, body, 0)
        process(DNG - 1, 1, False)
        for b in range(NBUF):
            scat(1, b).wait()
        idx_desc(0, DNG - 1).wait()
        plsc.subcore_barrier()

        pltpu.sync_copy(acc_sh.at[pl.ds(sid * ROWS_PER_SUB, ROWS_PER_SUB)],
                        cnt_hbm.at[pl.ds(cid * NP + sid * ROWS_PER_SUB,
                                         ROWS_PER_SUB)])

    return deg_kernel


_deg_kernel = _make_deg_kernel()


def _make_agg_kernel():
    mesh = plsc.VectorSubcoreMesh(core_axis_name="c", subcore_axis_name="s")

    H = NBUF // 2

    @functools.partial(
        pl.kernel,
        out_type=jax.ShapeDtypeStruct((2 * NP, 128), jnp.float32),
        mesh=mesh,
        scratch_types=[pltpu.VMEM((H, B), jnp.int32)] * 8
          + [pltpu.VMEM((B, 128), jnp.float32)] * NBUF
          + [pltpu.SemaphoreType.DMA] * (2 * NBUF + 2)
          + [pltpu.VMEM_SHARED((NP, 128), jnp.float32)],
    )
    def agg_kernel(src_hbm, dst_hbm, hp_hbm, out_hbm, *rest):
        idx = rest[:8]
        iset = (idx[0:4], idx[4:8])
        rows = rest[8:8 + NBUF]
        gsem = rest[8 + NBUF:8 + 2 * NBUF]
        ssem = rest[8 + 2 * NBUF:8 + 3 * NBUF]
        isem = rest[8 + 3 * NBUF:8 + 3 * NBUF + 2]
        acc_sh = rest[8 + 3 * NBUF + 2]
        cid = lax.axis_index("c")
        sid = lax.axis_index("s")
        rbase = sid * ROWS_PER_SUB

        pltpu.sync_copy(hp_hbm.at[pl.ds(cid * NP + rbase, ROWS_PER_SUB)],
                        acc_sh.at[pl.ds(rbase, ROWS_PER_SUB)])
        plsc.subcore_barrier()

        soff = cid * EROWS + sid * NB_EDGE
        doff = sid * NB_EDGE

        def idx_descs(p, g):
            sA, sB, dA, dB = iset[p]
            base = soff + g * NBUF
            dbase = doff + g * NBUF
            return [
                pltpu.make_async_copy(src_hbm.at[pl.ds(base, H)], sA, isem[p]),
                pltpu.make_async_copy(src_hbm.at[pl.ds(base + H, H)], sB, isem[p]),
                pltpu.make_async_copy(dst_hbm.at[pl.ds(dbase, H)], dA, isem[p]),
                pltpu.make_async_copy(dst_hbm.at[pl.ds(dbase + H, H)], dB, isem[p]),
            ]

        def scat(p, b):
            dref = iset[p][2] if b < H else iset[p][3]
            return pltpu.make_async_copy(
                rows[b], acc_sh.at[dref.at[b % H]], ssem[b])

        def process(g, p, first):
            sA, sB, dA, dB = iset[p]
            if not first:
                for b in range(H):
                    scat(1 - p, b).wait()
                for d in idx_descs(p, g):
                    d.wait()
            gdA = [pltpu.async_copy(hp_hbm.at[sA.at[b]], rows[b], gsem[b])
                   for b in range(H)]
            if not first:
                for b in range(H):
                    scat(1 - p, H + b).wait()
            gdB = [pltpu.async_copy(hp_hbm.at[sB.at[b]], rows[H + b],
                                    gsem[H + b]) for b in range(H)]
            gn = jnp.minimum(g + 1, NGRP - 1)
            for d in idx_descs(1 - p, gn):
                d.start()
            for b in range(H):
                gdA[b].wait()
                pltpu.async_copy(rows[b], acc_sh.at[dA.at[b]], ssem[b],
                                 add=True)
            for b in range(H):
                gdB[b].wait()
                pltpu.async_copy(rows[H + b], acc_sh.at[dB.at[b]],
                                 ssem[H + b], add=True)

        for d in idx_descs(0, 0):
            d.start()
        for d in idx_descs(0, 0):
            d.wait()
        process(0, 0, True)

        def body(tt, carry):
            g = 1 + tt * 2
            process(g, 1, False)
            process(g + 1, 0, False)
            return carry

        lax.fori_loop(0, (NGRP - 2) // 2, body, 0)
        process(NGRP - 1, 1, False)
        for b in range(NBUF):
            scat(1, b).wait()
        for d in idx_descs(0, NGRP - 1):
            d.wait()
        plsc.subcore_barrier()

        pltpu.sync_copy(acc_sh.at[pl.ds(rbase, ROWS_PER_SUB)],
                        out_hbm.at[pl.ds(cid * NP + rbase, ROWS_PER_SUB)])

    return agg_kernel


_agg_kernel = _make_agg_kernel()


def _dis(d0_ref, d1_ref):
    return lax.rsqrt(d0_ref[...] + d1_ref[...] + 1.0)


def _tc1_body(x_ref, w_ref, d0_ref, d1_ref, o_ref):
    o_ref[...] = jnp.dot(x_ref[...], w_ref[...],
                         preferred_element_type=jnp.float32) * _dis(d0_ref,
                                                                    d1_ref)


def _tc1(xp, W1, dis):
    return pl.pallas_call(
        _tc1_body,
        out_shape=jax.ShapeDtypeStruct((2 * NP, 128), jnp.float32),
        grid=(NC, NBLK),
        in_specs=[
            pl.BlockSpec((R, 128), lambda c, i: (i, 0)),
            pl.BlockSpec((128, 128), lambda c, i: (0, c)),
            pl.BlockSpec((R, 1), lambda c, i: (i, 0)),
            pl.BlockSpec((R, 1), lambda c, i: (NBLK + i, 0)),
        ],
        out_specs=pl.BlockSpec((R, 128), lambda c, i: (c * NBLK + i, 0)),
    )(xp, W1, dis, dis)


def _tcmid_body(s0_ref, s1_ref, d0_ref, d1_ref, b_ref, w0_ref, w1_ref, o_ref):
    dis = _dis(d0_ref, d1_ref)
    z0 = jnp.maximum(dis * s0_ref[...] + b_ref[0:1, :], 0.0)
    z1 = jnp.maximum(dis * s1_ref[...] + b_ref[1:2, :], 0.0)
    acc = jnp.dot(z0, w0_ref[...], preferred_element_type=jnp.float32)
    acc = acc + jnp.dot(z1, w1_ref[...], preferred_element_type=jnp.float32)
    o_ref[...] = acc * dis


def _tcmid(S, W, b, dis):
    b2d = b.reshape(2, 128)
    return pl.pallas_call(
        _tcmid_body,
        out_shape=jax.ShapeDtypeStruct((2 * NP, 128), jnp.float32),
        grid=(NC, NBLK),
        in_specs=[
            pl.BlockSpec((R, 128), lambda c, i: (i, 0)),
            pl.BlockSpec((R, 128), lambda c, i: (NBLK + i, 0)),
            pl.BlockSpec((R, 1), lambda c, i: (i, 0)),
            pl.BlockSpec((R, 1), lambda c, i: (NBLK + i, 0)),
            pl.BlockSpec((2, 128), lambda c, i: (0, 0)),
            pl.BlockSpec((128, 128), lambda c, i: (0, c)),
            pl.BlockSpec((128, 128), lambda c, i: (1, c)),
        ],
        out_specs=pl.BlockSpec((R, 128), lambda c, i: (c * NBLK + i, 0)),
    )(S, S, dis, dis, b2d, W, W)


def _tcfc_body(s0_ref, s1_ref, d0_ref, d1_ref, b_ref, w0_ref, w1_ref,
               bfc_ref, o_ref):
    dis = _dis(d0_ref, d1_ref)
    z0 = jnp.maximum(dis * s0_ref[...] + b_ref[0:1, :], 0.0)
    z1 = jnp.maximum(dis * s1_ref[...] + b_ref[1:2, :], 0.0)
    acc = jnp.dot(z0, w0_ref[...], preferred_element_type=jnp.float32)
    acc = acc + jnp.dot(z1, w1_ref[...], preferred_element_type=jnp.float32)
    o_ref[...] = acc + bfc_ref[...]


def _tcfc(S, Wfc, b, bfc, dis):
    b2d = b.reshape(2, 128)
    return pl.pallas_call(
        _tcfc_body,
        out_shape=jax.ShapeDtypeStruct((NP, 128), jnp.float32),
        grid=(NBLK,),
        in_specs=[
            pl.BlockSpec((R, 128), lambda i: (i, 0)),
            pl.BlockSpec((R, 128), lambda i: (NBLK + i, 0)),
            pl.BlockSpec((R, 1), lambda i: (i, 0)),
            pl.BlockSpec((R, 1), lambda i: (NBLK + i, 0)),
            pl.BlockSpec((2, 128), lambda i: (0, 0)),
            pl.BlockSpec((128, 128), lambda i: (0, 0)),
            pl.BlockSpec((128, 128), lambda i: (1, 0)),
            pl.BlockSpec((1, 128), lambda i: (0, 0)),
        ],
        out_specs=pl.BlockSpec((R, 128), lambda i: (i, 0)),
    )(S, S, dis, dis, b2d, Wfc, Wfc, bfc.reshape(1, 128))


def kernel(x, edge_index, W1, b1, W2, b2, W3, b3, Wfc, bfc):
    src = edge_index[0].astype(jnp.int32)
    dst = edge_index[1].astype(jnp.int32)
    pad = (jnp.arange(EP - E, dtype=jnp.int32) % 224) + 10016
    srcp = jnp.concatenate([src, pad])
    dstp = jnp.concatenate([dst, pad])
    src3 = jnp.concatenate([srcp, srcp + NP]).reshape(2 * EROWS, B)
    dst3 = dstp.reshape(EROWS, B)
    xp = jnp.zeros((NP, x.shape[1]), x.dtype).at[:N].set(x)

    dis = _deg_kernel(dst3).reshape(2 * NP, 1)
    hp = _tc1(xp, W1, dis)
    S = _agg_kernel(src3, dst3, hp)
    hp = _tcmid(S, W2, b1, dis)
    S = _agg_kernel(src3, dst3, hp)
    hp = _tcmid(S, W3, b2, dis)
    S = _agg_kernel(src3, dst3, hp)
    out = _tcfc(S, Wfc, b3, bfc, dis)
    return out[:N]


# speedup vs baseline: 1.1330x; 1.0078x over previous
import functools

import jax
import jax.numpy as jnp
from jax import lax
from jax.experimental import pallas as pl
from jax.experimental.pallas import tpu as pltpu
from jax.experimental.pallas import tpu_sc as plsc

N = 10000
NP = 10240
E = 320000
B = 80
NBUF = 4
NS = 16
NC = 2
EP = 327680
ROWS_PER_SUB = NP // NS
NB_EDGE = EP // (NS * B)
NGRP = NB_EDGE // NBUF
EROWS = EP // B
R = 5120
NBLK = NP // R


DNB = EROWS // (2 * NS)
DNG = DNB // NBUF


def _make_deg_kernel():
    mesh = plsc.VectorSubcoreMesh(core_axis_name="c", subcore_axis_name="s")

    @functools.partial(
        pl.kernel,
        out_type=jax.ShapeDtypeStruct((2 * NP,), jnp.float32),
        mesh=mesh,
        scratch_types=[
            pltpu.VMEM((NBUF, B), jnp.int32),
            pltpu.VMEM((NBUF, B), jnp.int32),
            pltpu.VMEM((B,), jnp.float32),
            pltpu.VMEM((ROWS_PER_SUB,), jnp.float32),
            pltpu.VMEM_SHARED((NP,), jnp.float32),
        ] + [pltpu.SemaphoreType.DMA] * (NBUF + 2),
    )
    def deg_kernel(dst_hbm, cnt_hbm, idx0, idx1, ones_v, z_v, acc_sh, *sems):
        ssem = sems[:NBUF]
        isem = sems[NBUF:NBUF + 2]
        iset = (idx0, idx1)
        cid = lax.axis_index("c")
        sid = lax.axis_index("s")

        for k in range(B // 16):
            ones_v[pl.ds(k * 16, 16)] = jnp.ones((16,), jnp.float32)
        for k in range(ROWS_PER_SUB // 16):
            z_v[pl.ds(k * 16, 16)] = jnp.zeros((16,), jnp.float32)
        pltpu.sync_copy(z_v, acc_sh.at[pl.ds(sid * ROWS_PER_SUB, ROWS_PER_SUB)])
        plsc.subcore_barrier()

        eoff = cid * (EROWS // 2) + sid * DNB

        def idx_desc(p, t):
            return pltpu.make_async_copy(
                dst_hbm.at[pl.ds(eoff + t * NBUF, NBUF)], iset[p], isem[p])

        def scat(p, b):
            return pltpu.make_async_copy(
                ones_v, acc_sh.at[iset[p].at[b]], ssem[b])

        def process(t, p, first):
            if not first:
                for b in range(NBUF):
                    scat(1 - p, b).wait()
                idx_desc(p, t).wait()
            tn = jnp.minimum(t + 1, DNG - 1)
            idx_desc(1 - p, tn).start()
            for b in range(NBUF):
                pltpu.async_copy(ones_v, acc_sh.at[iset[p].at[b]], ssem[b],
                                 add=True)

        idx_desc(0, 0).start()
        idx_desc(0, 0).wait()
        process(0, 0, True)

        def body(tt, carry):
            t = 1 + tt * 2
            process(t, 1, False)
            process(t + 1, 0, False)
            return carry

        lax.fori_loop(0, (DNG - 2) // 2, body, 0)
        process(DNG - 1, 1, False)
        for b in range(NBUF):
            scat(1, b).wait()
        idx_desc(0, DNG - 1).wait()
        plsc.subcore_barrier()

        pltpu.sync_copy(acc_sh.at[pl.ds(sid * ROWS_PER_SUB, ROWS_PER_SUB)],
                        cnt_hbm.at[pl.ds(cid * NP + sid * ROWS_PER_SUB,
                                         ROWS_PER_SUB)])

    return deg_kernel


_deg_kernel = _make_deg_kernel()


def _make_agg_kernel():
    mesh = plsc.VectorSubcoreMesh(core_axis_name="c", subcore_axis_name="s")

    H = NBUF // 2

    @functools.partial(
        pl.kernel,
        out_type=jax.ShapeDtypeStruct((2 * NP, 128), jnp.float32),
        mesh=mesh,
        scratch_types=[pltpu.VMEM((H, B), jnp.int32)] * 8
          + [pltpu.VMEM((B, 128), jnp.float32)] * NBUF
          + [pltpu.SemaphoreType.DMA] * (2 * NBUF + 2)
          + [pltpu.VMEM_SHARED((NP, 128), jnp.float32)],
    )
    def agg_kernel(src_hbm, dst_hbm, hp_hbm, out_hbm, *rest):
        idx = rest[:8]
        iset = (idx[0:4], idx[4:8])
        rows = rest[8:8 + NBUF]
        gsem = rest[8 + NBUF:8 + 2 * NBUF]
        ssem = rest[8 + 2 * NBUF:8 + 3 * NBUF]
        isem = rest[8 + 3 * NBUF:8 + 3 * NBUF + 2]
        acc_sh = rest[8 + 3 * NBUF + 2]
        cid = lax.axis_index("c")
        sid = lax.axis_index("s")
        rbase = sid * ROWS_PER_SUB

        pltpu.sync_copy(hp_hbm.at[pl.ds(cid * NP + rbase, ROWS_PER_SUB)],
                        acc_sh.at[pl.ds(rbase, ROWS_PER_SUB)])
        plsc.subcore_barrier()

        soff = cid * EROWS + sid * NB_EDGE
        doff = sid * NB_EDGE

        def idx_descs(p, g):
            sA, sB, dA, dB = iset[p]
            base = soff + g * NBUF
            dbase = doff + g * NBUF
            return [
                pltpu.make_async_copy(src_hbm.at[pl.ds(base, H)], sA, isem[p]),
                pltpu.make_async_copy(src_hbm.at[pl.ds(base + H, H)], sB, isem[p]),
                pltpu.make_async_copy(dst_hbm.at[pl.ds(dbase, H)], dA, isem[p]),
                pltpu.make_async_copy(dst_hbm.at[pl.ds(dbase + H, H)], dB, isem[p]),
            ]

        def scat(p, b):
            dref = iset[p][2] if b < H else iset[p][3]
            return pltpu.make_async_copy(
                rows[b], acc_sh.at[dref.at[b % H]], ssem[b])

        def process(g, p, first):
            sA, sB, dA, dB = iset[p]
            if not first:
                for b in range(H):
                    scat(1 - p, b).wait()
                for d in idx_descs(p, g):
                    d.wait()
            gdA = [pltpu.async_copy(hp_hbm.at[sA.at[b]], rows[b], gsem[b])
                   for b in range(H)]
            if not first:
                for b in range(H):
                    scat(1 - p, H + b).wait()
            gdB = [pltpu.async_copy(hp_hbm.at[sB.at[b]], rows[H + b],
                                    gsem[H + b]) for b in range(H)]
            gn = jnp.minimum(g + 1, NGRP - 1)
            for d in idx_descs(1 - p, gn):
                d.start()
            for b in range(H):
                gdA[b].wait()
                pltpu.async_copy(rows[b], acc_sh.at[dA.at[b]], ssem[b],
                                 add=True)
            for b in range(H):
                gdB[b].wait()
                pltpu.async_copy(rows[H + b], acc_sh.at[dB.at[b]],
                                 ssem[H + b], add=True)

        for d in idx_descs(0, 0):
            d.start()
        for d in idx_descs(0, 0):
            d.wait()
        process(0, 0, True)

        def body(tt, carry):
            g = 1 + tt * 2
            process(g, 1, False)
            process(g + 1, 0, False)
            return carry

        lax.fori_loop(0, (NGRP - 2) // 2, body, 0)
        process(NGRP - 1, 1, False)
        for b in range(NBUF):
            scat(1, b).wait()
        for d in idx_descs(0, NGRP - 1):
            d.wait()
        plsc.subcore_barrier()

        pltpu.sync_copy(acc_sh.at[pl.ds(rbase, ROWS_PER_SUB)],
                        out_hbm.at[pl.ds(cid * NP + rbase, ROWS_PER_SUB)])

    return agg_kernel


_agg_kernel = _make_agg_kernel()


def _dis(d0_ref, d1_ref):
    return lax.rsqrt(d0_ref[...] + d1_ref[...] + 1.0)


def _tc1_body(x_ref, w_ref, d0_ref, d1_ref, o_ref):
    o_ref[...] = jnp.dot(x_ref[...], w_ref[...],
                         preferred_element_type=jnp.float32) * _dis(d0_ref,
                                                                    d1_ref)


def _tc1(xp, W1, dis):
    return pl.pallas_call(
        _tc1_body,
        out_shape=jax.ShapeDtypeStruct((2 * NP, 128), jnp.float32),
        grid=(NC, NBLK),
        in_specs=[
            pl.BlockSpec((R, 128), lambda c, i: (i, 0)),
            pl.BlockSpec((128, 128), lambda c, i: (0, c)),
            pl.BlockSpec((R, 1), lambda c, i: (i, 0)),
            pl.BlockSpec((R, 1), lambda c, i: (NBLK + i, 0)),
        ],
        out_specs=pl.BlockSpec((R, 128), lambda c, i: (c * NBLK + i, 0)),
    )(xp, W1, dis, dis)


def _tcmid_body(s0_ref, s1_ref, d0_ref, d1_ref, b_ref, w0_ref, w1_ref, o_ref):
    dis = _dis(d0_ref, d1_ref)
    z0 = jnp.maximum(dis * s0_ref[...] + b_ref[0:1, :], 0.0)
    z1 = jnp.maximum(dis * s1_ref[...] + b_ref[1:2, :], 0.0)
    acc = jnp.dot(z0, w0_ref[...], preferred_element_type=jnp.float32)
    acc = acc + jnp.dot(z1, w1_ref[...], preferred_element_type=jnp.float32)
    o_ref[...] = acc * dis


def _tcmid(S, W, b, dis):
    b2d = b.reshape(2, 128)
    return pl.pallas_call(
        _tcmid_body,
        out_shape=jax.ShapeDtypeStruct((2 * NP, 128), jnp.float32),
        grid=(NC, NBLK),
        in_specs=[
            pl.BlockSpec((R, 128), lambda c, i: (i, 0)),
            pl.BlockSpec((R, 128), lambda c, i: (NBLK + i, 0)),
            pl.BlockSpec((R, 1), lambda c, i: (i, 0)),
            pl.BlockSpec((R, 1), lambda c, i: (NBLK + i, 0)),
            pl.BlockSpec((2, 128), lambda c, i: (0, 0)),
            pl.BlockSpec((128, 128), lambda c, i: (0, c)),
            pl.BlockSpec((128, 128), lambda c, i: (1, c)),
        ],
        out_specs=pl.BlockSpec((R, 128), lambda c, i: (c * NBLK + i, 0)),
    )(S, S, dis, dis, b2d, W, W)


def _tcfc_body(s0_ref, s1_ref, d0_ref, d1_ref, b_ref, w0_ref, w1_ref,
               bfc_ref, o_ref):
    dis = _dis(d0_ref, d1_ref)
    z0 = jnp.maximum(dis * s0_ref[...] + b_ref[0:1, :], 0.0)
    z1 = jnp.maximum(dis * s1_ref[...] + b_ref[1:2, :], 0.0)
    acc = jnp.dot(z0, w0_ref[...], preferred_element_type=jnp.float32)
    acc = acc + jnp.dot(z1, w1_ref[...], preferred_element_type=jnp.float32)
    o_ref[...] = acc + bfc_ref[...]


def _tcfc(S, Wfc, b, bfc, dis):
    b2d = b.reshape(2, 128)
    return pl.pallas_call(
        _tcfc_body,
        out_shape=jax.ShapeDtypeStruct((NP, 128), jnp.float32),
        grid=(NBLK,),
        in_specs=[
            pl.BlockSpec((R, 128), lambda i: (i, 0)),
            pl.BlockSpec((R, 128), lambda i: (NBLK + i, 0)),
            pl.BlockSpec((R, 1), lambda i: (i, 0)),
            pl.BlockSpec((R, 1), lambda i: (NBLK + i, 0)),
            pl.BlockSpec((2, 128), lambda i: (0, 0)),
            pl.BlockSpec((128, 128), lambda i: (0, 0)),
            pl.BlockSpec((128, 128), lambda i: (1, 0)),
            pl.BlockSpec((1, 128), lambda i: (0, 0)),
        ],
        out_specs=pl.BlockSpec((R, 128), lambda i: (i, 0)),
    )(S, S, dis, dis, b2d, Wfc, Wfc, bfc.reshape(1, 128))


def kernel(x, edge_index, W1, b1, W2, b2, W3, b3, Wfc, bfc):
    src = edge_index[0].astype(jnp.int32)
    dst = edge_index[1].astype(jnp.int32)
    pad = (jnp.arange(EP - E, dtype=jnp.int32) % 224) + 10016
    srcp = jnp.concatenate([src, pad])
    dstp = jnp.concatenate([dst, pad])
    src3 = jnp.concatenate([srcp, srcp + NP]).reshape(2 * EROWS, B)
    dst3 = dstp.reshape(EROWS, B)
    xp = jnp.zeros((NP, x.shape[1]), x.dtype).at[:N].set(x)

    dis = _deg_kernel(dst3).reshape(2 * NP, 1)
    hp = _tc1(xp, W1, dis)
    S = _agg_kernel(src3, dst3, hp)
    hp = _tcmid(S, W2, b1, dis)
    S = _agg_kernel(src3, dst3, hp)
    hp = _tcmid(S, W3, b2, dis)
    S = _agg_kernel(src3, dst3, hp)
    out = _tcfc(S, Wfc, b3, bfc, dis)
    return out[:N]


# speedup vs baseline: 1.1480x; 1.0133x over previous
import functools

import jax
import jax.numpy as jnp
from jax import lax
from jax.experimental import pallas as pl
from jax.experimental.pallas import tpu as pltpu
from jax.experimental.pallas import tpu_sc as plsc

N = 10000
NP = 10240
E = 320000
B = 80
NBUF = 4
NS = 16
NC = 2
EP = 327680
ROWS_PER_SUB = NP // NS
NB_EDGE = EP // (NS * B)
NGRP = NB_EDGE // NBUF
EROWS = EP // B
R = 10240
NBLK = NP // R


DNB = EROWS // (2 * NS)
DNG = DNB // NBUF


def _make_deg_kernel():
    mesh = plsc.VectorSubcoreMesh(core_axis_name="c", subcore_axis_name="s")

    @functools.partial(
        pl.kernel,
        out_type=jax.ShapeDtypeStruct((2 * NP,), jnp.float32),
        mesh=mesh,
        scratch_types=[
            pltpu.VMEM((NBUF, B), jnp.int32),
            pltpu.VMEM((NBUF, B), jnp.int32),
            pltpu.VMEM((B,), jnp.float32),
            pltpu.VMEM((ROWS_PER_SUB,), jnp.float32),
            pltpu.VMEM_SHARED((NP,), jnp.float32),
        ] + [pltpu.SemaphoreType.DMA] * (NBUF + 2),
    )
    def deg_kernel(dst_hbm, cnt_hbm, idx0, idx1, ones_v, z_v, acc_sh, *sems):
        ssem = sems[:NBUF]
        isem = sems[NBUF:NBUF + 2]
        iset = (idx0, idx1)
        cid = lax.axis_index("c")
        sid = lax.axis_index("s")

        for k in range(B // 16):
            ones_v[pl.ds(k * 16, 16)] = jnp.ones((16,), jnp.float32)
        for k in range(ROWS_PER_SUB // 16):
            z_v[pl.ds(k * 16, 16)] = jnp.zeros((16,), jnp.float32)
        pltpu.sync_copy(z_v, acc_sh.at[pl.ds(sid * ROWS_PER_SUB, ROWS_PER_SUB)])
        plsc.subcore_barrier()

        eoff = cid * (EROWS // 2) + sid * DNB

        def idx_desc(p, t):
            return pltpu.make_async_copy(
                dst_hbm.at[pl.ds(eoff + t * NBUF, NBUF)], iset[p], isem[p])

        def scat(p, b):
            return pltpu.make_async_copy(
                ones_v, acc_sh.at[iset[p].at[b]], ssem[b])

        def process(t, p, first):
            if not first:
                for b in range(NBUF):
                    scat(1 - p, b).wait()
                idx_desc(p, t).wait()
            tn = jnp.minimum(t + 1, DNG - 1)
            idx_desc(1 - p, tn).start()
            for b in range(NBUF):
                pltpu.async_copy(ones_v, acc_sh.at[iset[p].at[b]], ssem[b],
                                 add=True)

        idx_desc(0, 0).start()
        idx_desc(0, 0).wait()
        process(0, 0, True)

        def body(tt, carry):
            t = 1 + tt * 2
            process(t, 1, False)
            process(t + 1, 0, False)
            return carry

        lax.fori_loop(0, (DNG - 2) // 2, body, 0)
        process(DNG - 1, 1, False)
        for b in range(NBUF):
            scat(1, b).wait()
        idx_desc(0, DNG - 1).wait()
        plsc.subcore_barrier()

        pltpu.sync_copy(acc_sh.at[pl.ds(sid * ROWS_PER_SUB, ROWS_PER_SUB)],
                        cnt_hbm.at[pl.ds(cid * NP + sid * ROWS_PER_SUB,
                                         ROWS_PER_SUB)])

    return deg_kernel


_deg_kernel = _make_deg_kernel()


def _make_agg_kernel():
    mesh = plsc.VectorSubcoreMesh(core_axis_name="c", subcore_axis_name="s")

    H = NBUF // 2

    @functools.partial(
        pl.kernel,
        out_type=jax.ShapeDtypeStruct((2 * NP, 128), jnp.float32),
        mesh=mesh,
        scratch_types=[pltpu.VMEM((H, B), jnp.int32)] * 8
          + [pltpu.VMEM((B, 128), jnp.float32)] * NBUF
          + [pltpu.SemaphoreType.DMA] * (2 * NBUF + 2)
          + [pltpu.VMEM_SHARED((NP, 128), jnp.float32)],
    )
    def agg_kernel(src_hbm, dst_hbm, hp_hbm, out_hbm, *rest):
        idx = rest[:8]
        iset = (idx[0:4], idx[4:8])
        rows = rest[8:8 + NBUF]
        gsem = rest[8 + NBUF:8 + 2 * NBUF]
        ssem = rest[8 + 2 * NBUF:8 + 3 * NBUF]
        isem = rest[8 + 3 * NBUF:8 + 3 * NBUF + 2]
        acc_sh = rest[8 + 3 * NBUF + 2]
        cid = lax.axis_index("c")
        sid = lax.axis_index("s")
        rbase = sid * ROWS_PER_SUB

        pltpu.sync_copy(hp_hbm.at[pl.ds(cid * NP + rbase, ROWS_PER_SUB)],
                        acc_sh.at[pl.ds(rbase, ROWS_PER_SUB)])
        plsc.subcore_barrier()

        soff = cid * EROWS + sid * NB_EDGE
        doff = sid * NB_EDGE

        def idx_descs(p, g):
            sA, sB, dA, dB = iset[p]
            base = soff + g * NBUF
            dbase = doff + g * NBUF
            return [
                pltpu.make_async_copy(src_hbm.at[pl.ds(base, H)], sA, isem[p]),
                pltpu.make_async_copy(src_hbm.at[pl.ds(base + H, H)], sB, isem[p]),
                pltpu.make_async_copy(dst_hbm.at[pl.ds(dbase, H)], dA, isem[p]),
                pltpu.make_async_copy(dst_hbm.at[pl.ds(dbase + H, H)], dB, isem[p]),
            ]

        def scat(p, b):
            dref = iset[p][2] if b < H else iset[p][3]
            return pltpu.make_async_copy(
                rows[b], acc_sh.at[dref.at[b % H]], ssem[b])

        def process(g, p, first):
            sA, sB, dA, dB = iset[p]
            if not first:
                for b in range(H):
                    scat(1 - p, b).wait()
                for d in idx_descs(p, g):
                    d.wait()
            gdA = [pltpu.async_copy(hp_hbm.at[sA.at[b]], rows[b], gsem[b])
                   for b in range(H)]
            if not first:
                for b in range(H):
                    scat(1 - p, H + b).wait()
            gdB = [pltpu.async_copy(hp_hbm.at[sB.at[b]], rows[H + b],
                                    gsem[H + b]) for b in range(H)]
            gn = jnp.minimum(g + 1, NGRP - 1)
            for d in idx_descs(1 - p, gn):
                d.start()
            for b in range(H):
                gdA[b].wait()
                pltpu.async_copy(rows[b], acc_sh.at[dA.at[b]], ssem[b],
                                 add=True)
            for b in range(H):
                gdB[b].wait()
                pltpu.async_copy(rows[H + b], acc_sh.at[dB.at[b]],
                                 ssem[H + b], add=True)

        for d in idx_descs(0, 0):
            d.start()
        for d in idx_descs(0, 0):
            d.wait()
        process(0, 0, True)

        def body(tt, carry):
            g = 1 + tt * 2
            process(g, 1, False)
            process(g + 1, 0, False)
            return carry

        lax.fori_loop(0, (NGRP - 2) // 2, body, 0)
        process(NGRP - 1, 1, False)
        for b in range(NBUF):
            scat(1, b).wait()
        for d in idx_descs(0, NGRP - 1):
            d.wait()
        plsc.subcore_barrier()

        pltpu.sync_copy(acc_sh.at[pl.ds(rbase, ROWS_PER_SUB)],
                        out_hbm.at[pl.ds(cid * NP + rbase, ROWS_PER_SUB)])

    return agg_kernel


_agg_kernel = _make_agg_kernel()


def _dis(d0_ref, d1_ref):
    return lax.rsqrt(d0_ref[...] + d1_ref[...] + 1.0)


def _tc1_body(x_ref, w_ref, d0_ref, d1_ref, o_ref):
    o_ref[...] = jnp.dot(x_ref[...], w_ref[...],
                         preferred_element_type=jnp.float32) * _dis(d0_ref,
                                                                    d1_ref)


def _tc1(xp, W1, dis):
    return pl.pallas_call(
        _tc1_body,
        out_shape=jax.ShapeDtypeStruct((2 * NP, 128), jnp.float32),
        grid=(NC, NBLK),
        in_specs=[
            pl.BlockSpec((R, 128), lambda c, i: (i, 0)),
            pl.BlockSpec((128, 128), lambda c, i: (0, c)),
            pl.BlockSpec((R, 1), lambda c, i: (i, 0)),
            pl.BlockSpec((R, 1), lambda c, i: (NBLK + i, 0)),
        ],
        out_specs=pl.BlockSpec((R, 128), lambda c, i: (c * NBLK + i, 0)),
    )(xp, W1, dis, dis)


def _tcmid_body(s0_ref, s1_ref, d0_ref, d1_ref, b_ref, w0_ref, w1_ref, o_ref):
    dis = _dis(d0_ref, d1_ref)
    z0 = jnp.maximum(dis * s0_ref[...] + b_ref[0:1, :], 0.0)
    z1 = jnp.maximum(dis * s1_ref[...] + b_ref[1:2, :], 0.0)
    acc = jnp.dot(z0, w0_ref[...], preferred_element_type=jnp.float32)
    acc = acc + jnp.dot(z1, w1_ref[...], preferred_element_type=jnp.float32)
    o_ref[...] = acc * dis


def _tcmid(S, W, b, dis):
    b2d = b.reshape(2, 128)
    return pl.pallas_call(
        _tcmid_body,
        out_shape=jax.ShapeDtypeStruct((2 * NP, 128), jnp.float32),
        grid=(NC, NBLK),
        in_specs=[
            pl.BlockSpec((R, 128), lambda c, i: (i, 0)),
            pl.BlockSpec((R, 128), lambda c, i: (NBLK + i, 0)),
            pl.BlockSpec((R, 1), lambda c, i: (i, 0)),
            pl.BlockSpec((R, 1), lambda c, i: (NBLK + i, 0)),
            pl.BlockSpec((2, 128), lambda c, i: (0, 0)),
            pl.BlockSpec((128, 128), lambda c, i: (0, c)),
            pl.BlockSpec((128, 128), lambda c, i: (1, c)),
        ],
        out_specs=pl.BlockSpec((R, 128), lambda c, i: (c * NBLK + i, 0)),
    )(S, S, dis, dis, b2d, W, W)


def _tcfc_body(s0_ref, s1_ref, d0_ref, d1_ref, b_ref, w0_ref, w1_ref,
               bfc_ref, o_ref):
    dis = _dis(d0_ref, d1_ref)
    z0 = jnp.maximum(dis * s0_ref[...] + b_ref[0:1, :], 0.0)
    z1 = jnp.maximum(dis * s1_ref[...] + b_ref[1:2, :], 0.0)
    acc = jnp.dot(z0, w0_ref[...], preferred_element_type=jnp.float32)
    acc = acc + jnp.dot(z1, w1_ref[...], preferred_element_type=jnp.float32)
    o_ref[...] = acc + bfc_ref[...]


def _tcfc(S, Wfc, b, bfc, dis):
    b2d = b.reshape(2, 128)
    return pl.pallas_call(
        _tcfc_body,
        out_shape=jax.ShapeDtypeStruct((NP, 128), jnp.float32),
        grid=(NBLK,),
        in_specs=[
            pl.BlockSpec((R, 128), lambda i: (i, 0)),
            pl.BlockSpec((R, 128), lambda i: (NBLK + i, 0)),
            pl.BlockSpec((R, 1), lambda i: (i, 0)),
            pl.BlockSpec((R, 1), lambda i: (NBLK + i, 0)),
            pl.BlockSpec((2, 128), lambda i: (0, 0)),
            pl.BlockSpec((128, 128), lambda i: (0, 0)),
            pl.BlockSpec((128, 128), lambda i: (1, 0)),
            pl.BlockSpec((1, 128), lambda i: (0, 0)),
        ],
        out_specs=pl.BlockSpec((R, 128), lambda i: (i, 0)),
    )(S, S, dis, dis, b2d, Wfc, Wfc, bfc.reshape(1, 128))


def kernel(x, edge_index, W1, b1, W2, b2, W3, b3, Wfc, bfc):
    src = edge_index[0].astype(jnp.int32)
    dst = edge_index[1].astype(jnp.int32)
    pad = (jnp.arange(EP - E, dtype=jnp.int32) % 224) + 10016
    srcp = jnp.concatenate([src, pad])
    dstp = jnp.concatenate([dst, pad])
    src3 = jnp.concatenate([srcp, srcp + NP]).reshape(2 * EROWS, B)
    dst3 = dstp.reshape(EROWS, B)
    xp = jnp.zeros((NP, x.shape[1]), x.dtype).at[:N].set(x)

    dis = _deg_kernel(dst3).reshape(2 * NP, 1)
    hp = _tc1(xp, W1, dis)
    S = _agg_kernel(src3, dst3, hp)
    hp = _tcmid(S, W2, b1, dis)
    S = _agg_kernel(src3, dst3, hp)
    hp = _tcmid(S, W3, b2, dis)
    S = _agg_kernel(src3, dst3, hp)
    out = _tcfc(S, Wfc, b3, bfc, dis)
    return out[:N]
